# Initial kernel scaffold; baseline (speedup 1.0000x reference)
import functools

import jax
import jax.numpy as jnp
from jax import lax
from jax.experimental import pallas as pl
from jax.experimental.pallas import tpu as pltpu
from jax.experimental.pallas import tpu_sc as plsc

D = 1024
H = 16
DH = 64
E = 8
HID = 2048
S = 2048
VOCAB = 32000
L = 2
EPS = 1e-5

BT = 256
NPAD = 4096 + E * BT
NTILES = NPAD // BT

NW = 32
F32 = jnp.float32


def _sc_mesh():
    return plsc.VectorSubcoreMesh(core_axis_name="c", subcore_axis_name="s")


def _wid():
    return lax.axis_index("s") * 2 + lax.axis_index("c")


def _sc_gather(table, idx):
    n = idx.shape[0]
    per = n // NW

    @functools.partial(
        pl.kernel,
        out_type=jax.ShapeDtypeStruct((n, D), F32),
        mesh=_sc_mesh(),
        scratch_types=[
            pltpu.VMEM((per,), jnp.int32),
            pltpu.VMEM((per, D), F32),
            pltpu.SemaphoreType.DMA,
        ],
    )
    def k(table_hbm, idx_hbm, out_hbm, idx_v, rows_v, sem):
        base = _wid() * per
        pltpu.sync_copy(idx_hbm.at[pl.ds(base, per)], idx_v)
        pltpu.async_copy(table_hbm.at[idx_v], rows_v, sem).wait()
        pltpu.sync_copy(rows_v, out_hbm.at[pl.ds(base, per)])

    return k(table, idx)


def _sc_dispatch(x, d0, d1):
    per = S // NW

    @functools.partial(
        pl.kernel,
        out_type=jax.ShapeDtypeStruct((NPAD, D), F32),
        mesh=_sc_mesh(),
        scratch_types=[
            pltpu.VMEM((per,), jnp.int32),
            pltpu.VMEM((per,), jnp.int32),
            pltpu.VMEM((per, D), F32),
            pltpu.SemaphoreType.DMA,
        ],
    )
    def k(x_hbm, d0_hbm, d1_hbm, out_hbm, i0_v, i1_v, rows_v, sem):
        base = _wid() * per
        pltpu.sync_copy(x_hbm.at[pl.ds(base, per)], rows_v)
        pltpu.sync_copy(d0_hbm.at[pl.ds(base, per)], i0_v)
        pltpu.sync_copy(d1_hbm.at[pl.ds(base, per)], i1_v)
        pltpu.async_copy(rows_v, out_hbm.at[i0_v], sem).wait()
        pltpu.async_copy(rows_v, out_hbm.at[i1_v], sem).wait()

    return k(x, d0, d1)


def _sc_combine(ys, d0, d1):
    per = S // NW
    out = jax.ShapeDtypeStruct((S, D), F32)

    @functools.partial(
        pl.kernel,
        out_type=(out, out),
        mesh=_sc_mesh(),
        scratch_types=[
            pltpu.VMEM((per,), jnp.int32),
            pltpu.VMEM((per,), jnp.int32),
            pltpu.VMEM((per, D), F32),
            pltpu.SemaphoreType.DMA,
        ],
    )
    def k(ys_hbm, d0_hbm, d1_hbm, g0_hbm, g1_hbm, i0_v, i1_v, rows_v, sem):
        base = _wid() * per
        pltpu.sync_copy(d0_hbm.at[pl.ds(base, per)], i0_v)
        pltpu.sync_copy(d1_hbm.at[pl.ds(base, per)], i1_v)
        pltpu.async_copy(ys_hbm.at[i0_v], rows_v, sem).wait()
        pltpu.sync_copy(rows_v, g0_hbm.at[pl.ds(base, per)])
        pltpu.async_copy(ys_hbm.at[i1_v], rows_v, sem).wait()
        pltpu.sync_copy(rows_v, g1_hbm.at[pl.ds(base, per)])

    return k(ys, d0, d1)


def _dotT(a, b):
    return lax.dot_general(a, b, (((1,), (1,)), ((), ())),
                           preferred_element_type=F32)


def _layer_norm(h, g, b):
    m = jnp.mean(h, axis=-1, keepdims=True)
    v = jnp.mean((h - m) ** 2, axis=-1, keepdims=True)
    return (h - m) / jnp.sqrt(v + EPS) * g + b


def _qkv0(g, pos2d, W, b):
    def body(g_ref, p_ref, w_ref, b_ref, xp_ref, qkv_ref):
        xp = g_ref[...] + p_ref[...]
        xp_ref[...] = xp
        qkv_ref[...] = _dotT(xp, w_ref[...]) + b_ref[...]

    return pl.pallas_call(
        body,
        grid=(8, 3),
        in_specs=[
            pl.BlockSpec((S // 8, D), lambda i, j: (i, 0)),
            pl.BlockSpec((S // 8, D), lambda i, j: (i, 0)),
            pl.BlockSpec((D, D), lambda i, j: (j, 0)),
            pl.BlockSpec((1, D), lambda i, j: (0, j)),
        ],
        out_specs=[
            pl.BlockSpec((S // 8, D), lambda i, j: (i, 0)),
            pl.BlockSpec((S // 8, D), lambda i, j: (i, j)),
        ],
        out_shape=[
            jax.ShapeDtypeStruct((S, D), F32),
            jax.ShapeDtypeStruct((S, 3 * D), F32),
        ],
    )(g, pos2d, W, b.reshape(1, 3 * D))


def _qkv1(x, W, b):
    def body(x_ref, w_ref, b_ref, qkv_ref):
        qkv_ref[...] = _dotT(x_ref[...], w_ref[...]) + b_ref[...]

    return pl.pallas_call(
        body,
        grid=(8, 3),
        in_specs=[
            pl.BlockSpec((S // 8, D), lambda i, j: (i, 0)),
            pl.BlockSpec((D, D), lambda i, j: (j, 0)),
            pl.BlockSpec((1, D), lambda i, j: (0, j)),
        ],
        out_specs=pl.BlockSpec((S // 8, D), lambda i, j: (i, j)),
        out_shape=jax.ShapeDtypeStruct((S, 3 * D), F32),
    )(x, W, b.reshape(1, 3 * D))


def _attn(qkv):
    QB = 512

    def body(q_ref, k_ref, v_ref, o_ref):
        kk = k_ref[...]
        vv = v_ref[...]
        outs = []
        for hh in range(2):
            sl = slice(hh * DH, (hh + 1) * DH)
            s = _dotT(q_ref[:, sl], kk[:, sl]) * (1.0 / (DH ** 0.5))
            s = s - jnp.max(s, axis=-1, keepdims=True)
            p = jnp.exp(s)
            p = p / jnp.sum(p, axis=-1, keepdims=True)
            outs.append(lax.dot_general(p, vv[:, sl], (((1,), (0,)), ((), ())),
                                        preferred_element_type=F32))
        o_ref[...] = jnp.concatenate(outs, axis=1)

    return pl.pallas_call(
        body,
        grid=(8, S // QB),
        in_specs=[
            pl.BlockSpec((QB, 128), lambda hp, qb: (qb, hp)),
            pl.BlockSpec((S, 128), lambda hp, qb: (0, 8 + hp)),
            pl.BlockSpec((S, 128), lambda hp, qb: (0, 16 + hp)),
        ],
        out_specs=pl.BlockSpec((QB, 128), lambda hp, qb: (qb, hp)),
        out_shape=jax.ShapeDtypeStruct((S, D), F32),
    )(qkv, qkv, qkv)


def _postattn(ao, Wo, bo, xp, g, b):
    def body(ao_ref, wo_ref, bo_ref, xp_ref, g_ref, b_ref, o_ref):
        y = _dotT(ao_ref[...], wo_ref[...]) + bo_ref[...]
        h = xp_ref[...] + y
        o_ref[...] = _layer_norm(h, g_ref[...], b_ref[...])

    return pl.pallas_call(
        body,
        grid=(8,),
        in_specs=[
            pl.BlockSpec((S // 8, D), lambda i: (i, 0)),
            pl.BlockSpec((D, D), lambda i: (0, 0)),
            pl.BlockSpec((1, D), lambda i: (0, 0)),
            pl.BlockSpec((S // 8, D), lambda i: (i, 0)),
            pl.BlockSpec((1, D), lambda i: (0, 0)),
            pl.BlockSpec((1, D), lambda i: (0, 0)),
        ],
        out_specs=pl.BlockSpec((S // 8, D), lambda i: (i, 0)),
        out_shape=jax.ShapeDtypeStruct((S, D), F32),
    )(ao, Wo, bo.reshape(1, D), xp, g.reshape(1, D), b.reshape(1, D))


def _route(x1, Wr, br):
    def body(x_ref, wr_ref, br_ref, w_ref, d0_ref, d1_ref, eot_ref):
        x = x_ref[...]
        logits = _dotT(x, wr_ref[...]) + br_ref[...]
        mx = jnp.max(logits, axis=-1, keepdims=True)
        p = jnp.exp(logits - mx)
        gates = p / jnp.sum(p, axis=-1, keepdims=True)

        idx8 = lax.broadcasted_iota(jnp.int32, (S, E), 1)
        m1 = jnp.max(gates, axis=-1, keepdims=True)
        i1 = jnp.min(jnp.where(gates == m1, idx8, E), axis=-1, keepdims=True)
        oh0 = (idx8 == i1).astype(F32)
        masked = jnp.where(idx8 == i1, -1.0, gates)
        m2 = jnp.max(masked, axis=-1, keepdims=True)
        i2 = jnp.min(jnp.where(masked == m2, idx8, E), axis=-1, keepdims=True)
        oh1 = (idx8 == i2).astype(F32)

        wsum = m1 + m2
        w_ref[...] = jnp.concatenate([m1 / wsum, m2 / wsum], axis=1)

        ii = lax.broadcasted_iota(jnp.int32, (512, 512), 0)
        jj = lax.broadcasted_iota(jnp.int32, (512, 512), 1)
        tril = (jj < ii).astype(F32)

        def exc_prefix(oh):
            parts = []
            carry = jnp.zeros((1, E), F32)
            for bb in range(4):
                blk = oh[bb * 512:(bb + 1) * 512]
                parts.append(jnp.dot(tril, blk,
                                     preferred_element_type=F32) + carry)
                carry = carry + jnp.sum(blk, axis=0, keepdims=True)
            return jnp.concatenate(parts, axis=0), carry

        pre0, c0 = exc_prefix(oh0)
        pre1, c1 = exc_prefix(oh1)
        counts = c0 + c1
        padded = jnp.floor((counts + (BT - 1)) / BT) * BT
        ei = lax.broadcasted_iota(jnp.int32, (E, E), 0)
        ej = lax.broadcasted_iota(jnp.int32, (E, E), 1)
        tril8 = (ei < ej).astype(F32)
        pstart = jnp.dot(padded, tril8, preferred_element_type=F32)

        d0 = jnp.sum((pstart + pre0) * oh0, axis=-1, keepdims=True)
        d1 = jnp.sum((pstart + c0 + pre1) * oh1, axis=-1, keepdims=True)
        d0_ref[...] = d0.astype(jnp.int32)
        d1_ref[...] = d1.astype(jnp.int32)

        tstart = lax.broadcasted_iota(F32, (NTILES, E), 0) * BT
        cmp = (tstart >= pstart).astype(jnp.int32)
        eot_ref[...] = jnp.sum(cmp, axis=-1, keepdims=True) - 1

    return pl.pallas_call(
        body,
        grid=(1,),
        in_specs=[
            pl.BlockSpec((S, D), lambda i: (0, 0)),
            pl.BlockSpec((E, D), lambda i: (0, 0)),
            pl.BlockSpec((1, E), lambda i: (0, 0)),
        ],
        out_specs=[
            pl.BlockSpec((S, 2), lambda i: (0, 0)),
            pl.BlockSpec((S, 1), lambda i: (0, 0)),
            pl.BlockSpec((S, 1), lambda i: (0, 0)),
            pl.BlockSpec((NTILES, 1), lambda i: (0, 0)),
        ],
        out_shape=[
            jax.ShapeDtypeStruct((S, 2), F32),
            jax.ShapeDtypeStruct((S, 1), jnp.int32),
            jax.ShapeDtypeStruct((S, 1), jnp.int32),
            jax.ShapeDtypeStruct((NTILES, 1), jnp.int32),
        ],
    )(x1, Wr, br.reshape(1, E))


def _gmm(eot, xs, W1l, b1l, W2l, b2l):
    HB = 512

    def body(eot_ref, xs_ref, w1_ref, b1_ref, w2_ref, b2_ref, o_ref):
        j = pl.program_id(1)
        h = jnp.maximum(_dotT(xs_ref[...], w1_ref[0]) + b1_ref[0], 0.0)
        part = _dotT(h, w2_ref[0])

        @pl.when(j == 0)
        def _():
            o_ref[...] = part + b2_ref[0]

        @pl.when(j > 0)
        def _():
            o_ref[...] += part

    grid_spec = pltpu.PrefetchScalarGridSpec(
        num_scalar_prefetch=1,
        grid=(NTILES, HID // HB),
        in_specs=[
            pl.BlockSpec((BT, D), lambda i, j, eot: (i, 0)),
            pl.BlockSpec((1, HB, D), lambda i, j, eot: (eot[i, 0], j, 0)),
            pl.BlockSpec((1, HB), lambda i, j, eot: (eot[i, 0], j)),
            pl.BlockSpec((1, D, HB), lambda i, j, eot: (eot[i, 0], 0, j)),
            pl.BlockSpec((1, D), lambda i, j, eot: (eot[i, 0], 0)),
        ],
        out_specs=pl.BlockSpec((BT, D), lambda i, j, eot: (i, 0)),
    )
    return pl.pallas_call(
        body,
        grid_spec=grid_spec,
        out_shape=jax.ShapeDtypeStruct((NPAD, D), F32),
    )(eot, xs, W1l, b1l, W2l, b2l)


def _ln2(x1, g0, g1, w, g, b):
    def body(x_ref, g0_ref, g1_ref, w_ref, g_ref, b_ref, o_ref):
        ww = w_ref[...]
        moe = ww[:, 0:1] * g0_ref[...] + ww[:, 1:2] * g1_ref[...]
        h = x_ref[...] + moe
        o_ref[...] = _layer_norm(h, g_ref[...], b_ref[...])

    return pl.pallas_call(
        body,
        grid=(8,),
        in_specs=[
            pl.BlockSpec((S // 8, D), lambda i: (i, 0)),
            pl.BlockSpec((S // 8, D), lambda i: (i, 0)),
            pl.BlockSpec((S // 8, D), lambda i: (i, 0)),
            pl.BlockSpec((S // 8, 2), lambda i: (i, 0)),
            pl.BlockSpec((1, D), lambda i: (0, 0)),
            pl.BlockSpec((1, D), lambda i: (0, 0)),
        ],
        out_specs=pl.BlockSpec((S // 8, D), lambda i: (i, 0)),
        out_shape=jax.ShapeDtypeStruct((S, D), F32),
    )(x1, g0, g1, w, g.reshape(1, D), b.reshape(1, D))


def _final(x, Wout, bout):
    RB, CB = 512, 1280

    def body(x_ref, w_ref, b_ref, o_ref):
        o_ref[...] = _dotT(x_ref[...], w_ref[...]) + b_ref[...]

    return pl.pallas_call(
        body,
        grid=(S // RB, VOCAB // CB),
        in_specs=[
            pl.BlockSpec((RB, D), lambda i, j: (i, 0)),
            pl.BlockSpec((CB, D), lambda i, j: (j, 0)),
            pl.BlockSpec((1, CB), lambda i, j: (0, j)),
        ],
        out_specs=pl.BlockSpec((RB, CB), lambda i, j: (i, j)),
        out_shape=jax.ShapeDtypeStruct((S, VOCAB), F32),
    )(x, Wout, bout.reshape(1, VOCAB))


def kernel(src, emb, pos, Wqkv, bqkv, Wo, bo, ln1_g, ln1_b, ln2_g, ln2_b,
           Wr, br, W1, b1, W2, b2, Wout, bout):
    srcf = src.reshape(S)
    gath = _sc_gather(emb, srcf)
    pos2d = pos[0, :S, :]
    x = None
    for l in range(L):
        if l == 0:
            xp, qkv = _qkv0(gath, pos2d, Wqkv[l], bqkv[l])
        else:
            xp = x
            qkv = _qkv1(x, Wqkv[l], bqkv[l])
        ao = _attn(qkv)
        x1 = _postattn(ao, Wo[l], bo[l], xp, ln1_g[l], ln1_b[l])
        w, d0, d1, eot = _route(x1, Wr[l], br[l])
        d0f = d0.reshape(S)
        d1f = d1.reshape(S)
        xs = _sc_dispatch(x1, d0f, d1f)
        ys = _gmm(eot, xs, W1[l], b1[l], W2[l], b2[l])
        g0, g1 = _sc_combine(ys, d0f, d1f)
        x = _ln2(x1, g0, g1, w, ln2_g[l], ln2_b[l])
    out = _final(x, Wout, bout)
    return out.reshape(1, S, VOCAB)


# speedup vs baseline: 1.0439x; 1.0439x over previous
import functools

import jax
import jax.numpy as jnp
from jax import lax
from jax.experimental import pallas as pl
from jax.experimental.pallas import tpu as pltpu
from jax.experimental.pallas import tpu_sc as plsc

D = 1024
H = 16
DH = 64
E = 8
HID = 2048
S = 2048
VOCAB = 32000
L = 2
EPS = 1e-5

BT = 256
NPAD = 4096 + E * BT
NTILES = NPAD // BT

NW = 32
F32 = jnp.float32


def _sc_mesh():
    return plsc.VectorSubcoreMesh(core_axis_name="c", subcore_axis_name="s")


def _wid():
    return lax.axis_index("s") * 2 + lax.axis_index("c")


def _sc_gather(table, idx):
    n = idx.shape[0]
    per = n // NW

    @functools.partial(
        pl.kernel,
        out_type=jax.ShapeDtypeStruct((n, D), F32),
        mesh=_sc_mesh(),
        scratch_types=[
            pltpu.VMEM((per,), jnp.int32),
            pltpu.VMEM((per, D), F32),
            pltpu.SemaphoreType.DMA,
        ],
    )
    def k(table_hbm, idx_hbm, out_hbm, idx_v, rows_v, sem):
        base = _wid() * per
        pltpu.sync_copy(idx_hbm.at[pl.ds(base, per)], idx_v)
        pltpu.async_copy(table_hbm.at[idx_v], rows_v, sem).wait()
        pltpu.sync_copy(rows_v, out_hbm.at[pl.ds(base, per)])

    return k(table, idx)


def _sc_dispatch(x, d0, d1):
    per = S // NW

    @functools.partial(
        pl.kernel,
        out_type=jax.ShapeDtypeStruct((NPAD, D), F32),
        mesh=_sc_mesh(),
        scratch_types=[
            pltpu.VMEM((per,), jnp.int32),
            pltpu.VMEM((per,), jnp.int32),
            pltpu.VMEM((per, D), F32),
            pltpu.SemaphoreType.DMA,
        ],
    )
    def k(x_hbm, d0_hbm, d1_hbm, out_hbm, i0_v, i1_v, rows_v, sem):
        base = _wid() * per
        pltpu.sync_copy(x_hbm.at[pl.ds(base, per)], rows_v)
        pltpu.sync_copy(d0_hbm.at[pl.ds(base, per)], i0_v)
        pltpu.sync_copy(d1_hbm.at[pl.ds(base, per)], i1_v)
        pltpu.async_copy(rows_v, out_hbm.at[i0_v], sem).wait()
        pltpu.async_copy(rows_v, out_hbm.at[i1_v], sem).wait()

    return k(x, d0, d1)


def _sc_combine(ys, d0, d1):
    per = S // NW
    out = jax.ShapeDtypeStruct((S, D), F32)

    @functools.partial(
        pl.kernel,
        out_type=(out, out),
        mesh=_sc_mesh(),
        scratch_types=[
            pltpu.VMEM((per,), jnp.int32),
            pltpu.VMEM((per,), jnp.int32),
            pltpu.VMEM((per, D), F32),
            pltpu.SemaphoreType.DMA,
        ],
    )
    def k(ys_hbm, d0_hbm, d1_hbm, g0_hbm, g1_hbm, i0_v, i1_v, rows_v, sem):
        base = _wid() * per
        pltpu.sync_copy(d0_hbm.at[pl.ds(base, per)], i0_v)
        pltpu.sync_copy(d1_hbm.at[pl.ds(base, per)], i1_v)
        pltpu.async_copy(ys_hbm.at[i0_v], rows_v, sem).wait()
        pltpu.sync_copy(rows_v, g0_hbm.at[pl.ds(base, per)])
        pltpu.async_copy(ys_hbm.at[i1_v], rows_v, sem).wait()
        pltpu.sync_copy(rows_v, g1_hbm.at[pl.ds(base, per)])

    return k(ys, d0, d1)


BF16 = jnp.bfloat16


def _dotT(a, b):
    return lax.dot_general(a.astype(BF16), b.astype(BF16),
                           (((1,), (1,)), ((), ())),
                           preferred_element_type=F32)


def _layer_norm(h, g, b):
    m = jnp.mean(h, axis=-1, keepdims=True)
    v = jnp.mean((h - m) ** 2, axis=-1, keepdims=True)
    return (h - m) / jnp.sqrt(v + EPS) * g + b


def _qkv0(g, pos2d, W, b):
    def body(g_ref, p_ref, w_ref, b_ref, xp_ref, qkv_ref):
        xp = g_ref[...] + p_ref[...]
        xp_ref[...] = xp
        qkv_ref[...] = _dotT(xp, w_ref[...]) + b_ref[...]

    return pl.pallas_call(
        body,
        grid=(8, 3),
        in_specs=[
            pl.BlockSpec((S // 8, D), lambda i, j: (i, 0)),
            pl.BlockSpec((S // 8, D), lambda i, j: (i, 0)),
            pl.BlockSpec((D, D), lambda i, j: (j, 0)),
            pl.BlockSpec((1, D), lambda i, j: (0, j)),
        ],
        out_specs=[
            pl.BlockSpec((S // 8, D), lambda i, j: (i, 0)),
            pl.BlockSpec((S // 8, D), lambda i, j: (i, j)),
        ],
        out_shape=[
            jax.ShapeDtypeStruct((S, D), F32),
            jax.ShapeDtypeStruct((S, 3 * D), F32),
        ],
    )(g, pos2d, W, b.reshape(1, 3 * D))


def _qkv1(x, W, b):
    def body(x_ref, w_ref, b_ref, qkv_ref):
        qkv_ref[...] = _dotT(x_ref[...], w_ref[...]) + b_ref[...]

    return pl.pallas_call(
        body,
        grid=(8, 3),
        in_specs=[
            pl.BlockSpec((S // 8, D), lambda i, j: (i, 0)),
            pl.BlockSpec((D, D), lambda i, j: (j, 0)),
            pl.BlockSpec((1, D), lambda i, j: (0, j)),
        ],
        out_specs=pl.BlockSpec((S // 8, D), lambda i, j: (i, j)),
        out_shape=jax.ShapeDtypeStruct((S, 3 * D), F32),
    )(x, W, b.reshape(1, 3 * D))


def _attn(qkv):
    QB = 512

    def body(q_ref, k_ref, v_ref, o_ref):
        kk = k_ref[...]
        vv = v_ref[...]
        outs = []
        for hh in range(2):
            sl = slice(hh * DH, (hh + 1) * DH)
            s = _dotT(q_ref[:, sl], kk[:, sl]) * (1.0 / (DH ** 0.5))
            s = s - jnp.max(s, axis=-1, keepdims=True)
            p = jnp.exp(s)
            p = p / jnp.sum(p, axis=-1, keepdims=True)
            outs.append(lax.dot_general(p.astype(BF16), vv[:, sl].astype(BF16),
                                        (((1,), (0,)), ((), ())),
                                        preferred_element_type=F32))
        o_ref[...] = jnp.concatenate(outs, axis=1)

    return pl.pallas_call(
        body,
        grid=(8, S // QB),
        in_specs=[
            pl.BlockSpec((QB, 128), lambda hp, qb: (qb, hp)),
            pl.BlockSpec((S, 128), lambda hp, qb: (0, 8 + hp)),
            pl.BlockSpec((S, 128), lambda hp, qb: (0, 16 + hp)),
        ],
        out_specs=pl.BlockSpec((QB, 128), lambda hp, qb: (qb, hp)),
        out_shape=jax.ShapeDtypeStruct((S, D), F32),
    )(qkv, qkv, qkv)


def _postattn(ao, Wo, bo, xp, g, b):
    def body(ao_ref, wo_ref, bo_ref, xp_ref, g_ref, b_ref, o_ref):
        y = _dotT(ao_ref[...], wo_ref[...]) + bo_ref[...]
        h = xp_ref[...] + y
        o_ref[...] = _layer_norm(h, g_ref[...], b_ref[...])

    return pl.pallas_call(
        body,
        grid=(8,),
        in_specs=[
            pl.BlockSpec((S // 8, D), lambda i: (i, 0)),
            pl.BlockSpec((D, D), lambda i: (0, 0)),
            pl.BlockSpec((1, D), lambda i: (0, 0)),
            pl.BlockSpec((S // 8, D), lambda i: (i, 0)),
            pl.BlockSpec((1, D), lambda i: (0, 0)),
            pl.BlockSpec((1, D), lambda i: (0, 0)),
        ],
        out_specs=pl.BlockSpec((S // 8, D), lambda i: (i, 0)),
        out_shape=jax.ShapeDtypeStruct((S, D), F32),
    )(ao, Wo, bo.reshape(1, D), xp, g.reshape(1, D), b.reshape(1, D))


def _route(x1, Wr, br):
    def body(x_ref, wr_ref, br_ref, w_ref, d0_ref, d1_ref, eot_ref):
        x = x_ref[...]
        logits = _dotT(x, wr_ref[...]) + br_ref[...]
        mx = jnp.max(logits, axis=-1, keepdims=True)
        p = jnp.exp(logits - mx)
        gates = p / jnp.sum(p, axis=-1, keepdims=True)

        idx8 = lax.broadcasted_iota(jnp.int32, (S, E), 1)
        m1 = jnp.max(gates, axis=-1, keepdims=True)
        i1 = jnp.min(jnp.where(gates == m1, idx8, E), axis=-1, keepdims=True)
        oh0 = (idx8 == i1).astype(F32)
        masked = jnp.where(idx8 == i1, -1.0, gates)
        m2 = jnp.max(masked, axis=-1, keepdims=True)
        i2 = jnp.min(jnp.where(masked == m2, idx8, E), axis=-1, keepdims=True)
        oh1 = (idx8 == i2).astype(F32)

        wsum = m1 + m2
        w_ref[...] = jnp.concatenate([m1 / wsum, m2 / wsum], axis=1)

        ii = lax.broadcasted_iota(jnp.int32, (512, 512), 0)
        jj = lax.broadcasted_iota(jnp.int32, (512, 512), 1)
        tril = (jj < ii).astype(F32)

        def exc_prefix(oh):
            parts = []
            carry = jnp.zeros((1, E), F32)
            for bb in range(4):
                blk = oh[bb * 512:(bb + 1) * 512]
                parts.append(jnp.dot(tril, blk, preferred_element_type=F32,
                                     precision=lax.Precision.HIGHEST) + carry)
                carry = carry + jnp.sum(blk, axis=0, keepdims=True)
            return jnp.concatenate(parts, axis=0), carry

        pre0, c0 = exc_prefix(oh0)
        pre1, c1 = exc_prefix(oh1)
        counts = c0 + c1
        padded = jnp.floor((counts + (BT - 1)) / BT) * BT
        ei = lax.broadcasted_iota(jnp.int32, (E, E), 0)
        ej = lax.broadcasted_iota(jnp.int32, (E, E), 1)
        tril8 = (ei < ej).astype(F32)
        pstart = jnp.dot(padded, tril8, preferred_element_type=F32,
                         precision=lax.Precision.HIGHEST)

        d0 = jnp.sum((pstart + pre0) * oh0, axis=-1, keepdims=True)
        d1 = jnp.sum((pstart + c0 + pre1) * oh1, axis=-1, keepdims=True)
        d0_ref[...] = d0.astype(jnp.int32)
        d1_ref[...] = d1.astype(jnp.int32)

        tstart = lax.broadcasted_iota(jnp.int32, (NTILES, E), 0).astype(F32) * BT
        cmp = (tstart >= pstart).astype(jnp.int32)
        eot_ref[...] = jnp.sum(cmp, axis=-1, keepdims=True) - 1

    return pl.pallas_call(
        body,
        grid=(1,),
        in_specs=[
            pl.BlockSpec((S, D), lambda i: (0, 0)),
            pl.BlockSpec((E, D), lambda i: (0, 0)),
            pl.BlockSpec((1, E), lambda i: (0, 0)),
        ],
        out_specs=[
            pl.BlockSpec((S, 2), lambda i: (0, 0)),
            pl.BlockSpec((S, 1), lambda i: (0, 0)),
            pl.BlockSpec((S, 1), lambda i: (0, 0)),
            pl.BlockSpec((NTILES, 1), lambda i: (0, 0)),
        ],
        out_shape=[
            jax.ShapeDtypeStruct((S, 2), F32),
            jax.ShapeDtypeStruct((S, 1), jnp.int32),
            jax.ShapeDtypeStruct((S, 1), jnp.int32),
            jax.ShapeDtypeStruct((NTILES, 1), jnp.int32),
        ],
    )(x1, Wr, br.reshape(1, E))


def _gmm(eot, xs, W1l, b1l, W2l, b2l):
    HB = 512

    def body(eot_ref, xs_ref, w1_ref, b1_ref, w2_ref, b2_ref, o_ref):
        j = pl.program_id(1)
        h = jnp.maximum(_dotT(xs_ref[...], w1_ref[0]) + b1_ref[0], 0.0)
        part = _dotT(h, w2_ref[0])

        @pl.when(j == 0)
        def _():
            o_ref[...] = part + b2_ref[0]

        @pl.when(j > 0)
        def _():
            o_ref[...] += part

    grid_spec = pltpu.PrefetchScalarGridSpec(
        num_scalar_prefetch=1,
        grid=(NTILES, HID // HB),
        in_specs=[
            pl.BlockSpec((BT, D), lambda i, j, eot: (i, 0)),
            pl.BlockSpec((1, HB, D), lambda i, j, eot: (eot[i, 0], j, 0)),
            pl.BlockSpec((1, 1, HB), lambda i, j, eot: (eot[i, 0], 0, j)),
            pl.BlockSpec((1, D, HB), lambda i, j, eot: (eot[i, 0], 0, j)),
            pl.BlockSpec((1, 1, D), lambda i, j, eot: (eot[i, 0], 0, 0)),
        ],
        out_specs=pl.BlockSpec((BT, D), lambda i, j, eot: (i, 0)),
    )
    return pl.pallas_call(
        body,
        grid_spec=grid_spec,
        out_shape=jax.ShapeDtypeStruct((NPAD, D), F32),
    )(eot, xs, W1l, b1l.reshape(E, 1, HID), W2l, b2l.reshape(E, 1, D))


def _ln2(x1, g0, g1, w, g, b):
    def body(x_ref, g0_ref, g1_ref, w_ref, g_ref, b_ref, o_ref):
        ww = w_ref[...]
        moe = ww[:, 0:1] * g0_ref[...] + ww[:, 1:2] * g1_ref[...]
        h = x_ref[...] + moe
        o_ref[...] = _layer_norm(h, g_ref[...], b_ref[...])

    return pl.pallas_call(
        body,
        grid=(8,),
        in_specs=[
            pl.BlockSpec((S // 8, D), lambda i: (i, 0)),
            pl.BlockSpec((S // 8, D), lambda i: (i, 0)),
            pl.BlockSpec((S // 8, D), lambda i: (i, 0)),
            pl.BlockSpec((S // 8, 2), lambda i: (i, 0)),
            pl.BlockSpec((1, D), lambda i: (0, 0)),
            pl.BlockSpec((1, D), lambda i: (0, 0)),
        ],
        out_specs=pl.BlockSpec((S // 8, D), lambda i: (i, 0)),
        out_shape=jax.ShapeDtypeStruct((S, D), F32),
    )(x1, g0, g1, w, g.reshape(1, D), b.reshape(1, D))


def _final(x, Wout, bout):
    RB, CB = 512, 1280

    def body(x_ref, w_ref, b_ref, o_ref):
        o_ref[...] = _dotT(x_ref[...], w_ref[...]) + b_ref[...]

    return pl.pallas_call(
        body,
        grid=(S // RB, VOCAB // CB),
        in_specs=[
            pl.BlockSpec((RB, D), lambda i, j: (i, 0)),
            pl.BlockSpec((CB, D), lambda i, j: (j, 0)),
            pl.BlockSpec((1, CB), lambda i, j: (0, j)),
        ],
        out_specs=pl.BlockSpec((RB, CB), lambda i, j: (i, j)),
        out_shape=jax.ShapeDtypeStruct((S, VOCAB), F32),
    )(x, Wout, bout.reshape(1, VOCAB))


def kernel(src, emb, pos, Wqkv, bqkv, Wo, bo, ln1_g, ln1_b, ln2_g, ln2_b,
           Wr, br, W1, b1, W2, b2, Wout, bout):
    Wqkv = Wqkv.astype(BF16)
    Wo = Wo.astype(BF16)
    W1 = W1.astype(BF16)
    W2 = W2.astype(BF16)
    Wout = Wout.astype(BF16)
    srcf = src.reshape(S)
    gath = _sc_gather(emb, srcf)
    pos2d = pos[0, :S, :]
    x = None
    for l in range(L):
        if l == 0:
            xp, qkv = _qkv0(gath, pos2d, Wqkv[l], bqkv[l])
        else:
            xp = x
            qkv = _qkv1(x, Wqkv[l], bqkv[l])
        ao = _attn(qkv)
        x1 = _postattn(ao, Wo[l], bo[l], xp, ln1_g[l], ln1_b[l])
        w, d0, d1, eot = _route(x1, Wr[l], br[l])
        d0f = d0.reshape(S)
        d1f = d1.reshape(S)
        xs = _sc_dispatch(x1, d0f, d1f)
        ys = _gmm(eot, xs, W1[l], b1[l], W2[l], b2[l])
        g0, g1 = _sc_combine(ys, d0f, d1f)
        x = _ln2(x1, g0, g1, w, ln2_g[l], ln2_b[l])
    out = _final(x, Wout, bout)
    return out.reshape(1, S, VOCAB)


# speedup vs baseline: 1.0695x; 1.0246x over previous
import functools

import jax
import jax.numpy as jnp
from jax import lax
from jax.experimental import pallas as pl
from jax.experimental.pallas import tpu as pltpu
from jax.experimental.pallas import tpu_sc as plsc

D = 1024
H = 16
DH = 64
E = 8
HID = 2048
S = 2048
VOCAB = 32000
L = 2
EPS = 1e-5

BT = 256
NPAD = 4096 + E * BT
NTILES = NPAD // BT

NW = 32
F32 = jnp.float32
BF16 = jnp.bfloat16


def _sc_mesh():
    return plsc.VectorSubcoreMesh(core_axis_name="c", subcore_axis_name="s")


def _wid():
    return lax.axis_index("s") * 2 + lax.axis_index("c")


def _sc_gather(table, idx):
    n = idx.shape[0]
    per = n // NW

    @functools.partial(
        pl.kernel,
        out_type=jax.ShapeDtypeStruct((n, D), F32),
        mesh=_sc_mesh(),
        scratch_types=[
            pltpu.VMEM((per,), jnp.int32),
            pltpu.VMEM((per, D), F32),
            pltpu.SemaphoreType.DMA,
        ],
    )
    def k(table_hbm, idx_hbm, out_hbm, idx_v, rows_v, sem):
        base = _wid() * per
        pltpu.sync_copy(idx_hbm.at[pl.ds(base, per)], idx_v)
        pltpu.async_copy(table_hbm.at[idx_v], rows_v, sem).wait()
        pltpu.sync_copy(rows_v, out_hbm.at[pl.ds(base, per)])

    return k(table, idx)


def _sc_dispatch(x, d0, d1):
    per = S // NW

    @functools.partial(
        pl.kernel,
        out_type=jax.ShapeDtypeStruct((NPAD, D), F32),
        mesh=_sc_mesh(),
        scratch_types=[
            pltpu.VMEM((per,), jnp.int32),
            pltpu.VMEM((per,), jnp.int32),
            pltpu.VMEM((per, D), F32),
            pltpu.SemaphoreType.DMA,
            pltpu.SemaphoreType.DMA,
        ],
    )
    def k(x_hbm, d0_hbm, d1_hbm, out_hbm, i0_v, i1_v, rows_v, sem0, sem1):
        base = _wid() * per
        pltpu.sync_copy(x_hbm.at[pl.ds(base, per)], rows_v)
        pltpu.sync_copy(d0_hbm.at[pl.ds(base, per)], i0_v)
        pltpu.sync_copy(d1_hbm.at[pl.ds(base, per)], i1_v)
        c0 = pltpu.async_copy(rows_v, out_hbm.at[i0_v], sem0)
        c1 = pltpu.async_copy(rows_v, out_hbm.at[i1_v], sem1)
        c0.wait()
        c1.wait()

    return k(x, d0, d1)


def _sc_combine(ys, d0, d1):
    per = S // NW
    out = jax.ShapeDtypeStruct((S, D), F32)
    hc = per // 2

    @functools.partial(
        pl.kernel,
        out_type=(out, out),
        mesh=_sc_mesh(),
        scratch_types=[
            pltpu.VMEM((per,), jnp.int32),
            pltpu.VMEM((per,), jnp.int32),
            pltpu.VMEM((hc, D), F32),
            pltpu.VMEM((hc, D), F32),
            pltpu.SemaphoreType.DMA,
            pltpu.SemaphoreType.DMA,
            pltpu.SemaphoreType.DMA,
            pltpu.SemaphoreType.DMA,
        ],
    )
    def k(ys_hbm, d0_hbm, d1_hbm, g0_hbm, g1_hbm, i0_v, i1_v, bufa, bufb,
          sg0, sg1, sw0, sw1):
        base = _wid() * per
        pltpu.sync_copy(d0_hbm.at[pl.ds(base, per)], i0_v)
        pltpu.sync_copy(d1_hbm.at[pl.ds(base, per)], i1_v)
        ga = pltpu.async_copy(ys_hbm.at[i0_v.at[pl.ds(0, hc)]], bufa, sg0)
        gb = pltpu.async_copy(ys_hbm.at[i0_v.at[pl.ds(hc, hc)]], bufb, sg1)
        ga.wait()
        wa = pltpu.async_copy(bufa, g0_hbm.at[pl.ds(base, hc)], sw0)
        gb.wait()
        wb = pltpu.async_copy(bufb, g0_hbm.at[pl.ds(base + hc, hc)], sw1)
        wa.wait()
        gc = pltpu.async_copy(ys_hbm.at[i1_v.at[pl.ds(0, hc)]], bufa, sg0)
        wb.wait()
        gd = pltpu.async_copy(ys_hbm.at[i1_v.at[pl.ds(hc, hc)]], bufb, sg1)
        gc.wait()
        wc = pltpu.async_copy(bufa, g1_hbm.at[pl.ds(base, hc)], sw0)
        gd.wait()
        wd = pltpu.async_copy(bufb, g1_hbm.at[pl.ds(base + hc, hc)], sw1)
        wc.wait()
        wd.wait()

    return k(ys, d0, d1)


def _dotT(a, b):
    return lax.dot_general(a.astype(BF16), b.astype(BF16),
                           (((1,), (1,)), ((), ())),
                           preferred_element_type=F32)


def _layer_norm(h, g, b):
    m = jnp.mean(h, axis=-1, keepdims=True)
    v = jnp.mean((h - m) ** 2, axis=-1, keepdims=True)
    return (h - m) / jnp.sqrt(v + EPS) * g + b


def _qkv0(g, pos2d, W, b):
    def body(g_ref, p_ref, w_ref, b_ref, xp_ref, qkv_ref):
        xp = g_ref[...] + p_ref[...]
        xp_ref[...] = xp
        qkv_ref[...] = _dotT(xp, w_ref[...]) + b_ref[...]

    return pl.pallas_call(
        body,
        grid=(8, 3),
        in_specs=[
            pl.BlockSpec((S // 8, D), lambda i, j: (i, 0)),
            pl.BlockSpec((S // 8, D), lambda i, j: (i, 0)),
            pl.BlockSpec((D, D), lambda i, j: (j, 0)),
            pl.BlockSpec((1, D), lambda i, j: (0, j)),
        ],
        out_specs=[
            pl.BlockSpec((S // 8, D), lambda i, j: (i, 0)),
            pl.BlockSpec((S // 8, D), lambda i, j: (i, j)),
        ],
        out_shape=[
            jax.ShapeDtypeStruct((S, D), F32),
            jax.ShapeDtypeStruct((S, 3 * D), F32),
        ],
    )(g, pos2d, W, b.reshape(1, 3 * D))


def _ln2qkv(x1, g0, g1, w, lg, lb, W, b):
    def body(x_ref, g0_ref, g1_ref, w_ref, lg_ref, lb_ref, wq_ref, b_ref,
             x2_ref, qkv_ref):
        j = pl.program_id(1)

        @pl.when(j == 0)
        def _():
            ww = w_ref[...]
            moe = ww[:, 0:1] * g0_ref[...] + ww[:, 1:2] * g1_ref[...]
            x2_ref[...] = _layer_norm(x_ref[...] + moe, lg_ref[...],
                                      lb_ref[...])

        qkv_ref[...] = _dotT(x2_ref[...], wq_ref[...]) + b_ref[...]

    return pl.pallas_call(
        body,
        grid=(8, 3),
        in_specs=[
            pl.BlockSpec((S // 8, D), lambda i, j: (i, 0)),
            pl.BlockSpec((S // 8, D), lambda i, j: (i, 0)),
            pl.BlockSpec((S // 8, D), lambda i, j: (i, 0)),
            pl.BlockSpec((S // 8, 2), lambda i, j: (i, 0)),
            pl.BlockSpec((1, D), lambda i, j: (0, 0)),
            pl.BlockSpec((1, D), lambda i, j: (0, 0)),
            pl.BlockSpec((D, D), lambda i, j: (j, 0)),
            pl.BlockSpec((1, D), lambda i, j: (0, j)),
        ],
        out_specs=[
            pl.BlockSpec((S // 8, D), lambda i, j: (i, 0)),
            pl.BlockSpec((S // 8, D), lambda i, j: (i, j)),
        ],
        out_shape=[
            jax.ShapeDtypeStruct((S, D), F32),
            jax.ShapeDtypeStruct((S, 3 * D), F32),
        ],
    )(x1, g0, g1, w, lg.reshape(1, D), lb.reshape(1, D), W,
      b.reshape(1, 3 * D))


def _attn(qkv):
    QB = 1024

    def body(q_ref, k_ref, v_ref, o_ref):
        kk = k_ref[...]
        vv = v_ref[...]
        outs = []
        for hh in range(2):
            sl = slice(hh * DH, (hh + 1) * DH)
            s = _dotT(q_ref[:, sl], kk[:, sl]) * (1.0 / (DH ** 0.5))
            s = s - jnp.max(s, axis=-1, keepdims=True)
            p = jnp.exp(s)
            p = p / jnp.sum(p, axis=-1, keepdims=True)
            outs.append(lax.dot_general(p.astype(BF16), vv[:, sl].astype(BF16),
                                        (((1,), (0,)), ((), ())),
                                        preferred_element_type=F32))
        o_ref[...] = jnp.concatenate(outs, axis=1)

    return pl.pallas_call(
        body,
        grid=(8, S // QB),
        in_specs=[
            pl.BlockSpec((QB, 128), lambda hp, qb: (qb, hp)),
            pl.BlockSpec((S, 128), lambda hp, qb: (0, 8 + hp)),
            pl.BlockSpec((S, 128), lambda hp, qb: (0, 16 + hp)),
        ],
        out_specs=pl.BlockSpec((QB, 128), lambda hp, qb: (qb, hp)),
        out_shape=jax.ShapeDtypeStruct((S, D), F32),
    )(qkv, qkv, qkv)


def _postattn_route(ao, Wo, bo, xp, lg, lb, Wr, br):
    def body(ao_ref, wo_ref, bo_ref, xp_ref, lg_ref, lb_ref, wr_ref, br_ref,
             x1_ref, w_ref, d0_ref, d1_ref, eot_ref):
        y = _dotT(ao_ref[...], wo_ref[...]) + bo_ref[...]
        x = _layer_norm(xp_ref[...] + y, lg_ref[...], lb_ref[...])
        x1_ref[...] = x

        logits = _dotT(x, wr_ref[...]) + br_ref[...]
        mx = jnp.max(logits, axis=-1, keepdims=True)
        p = jnp.exp(logits - mx)
        gates = p / jnp.sum(p, axis=-1, keepdims=True)

        idx8 = lax.broadcasted_iota(jnp.int32, (S, E), 1)
        m1 = jnp.max(gates, axis=-1, keepdims=True)
        i1 = jnp.min(jnp.where(gates == m1, idx8, E), axis=-1, keepdims=True)
        oh0 = (idx8 == i1).astype(F32)
        masked = jnp.where(idx8 == i1, -1.0, gates)
        m2 = jnp.max(masked, axis=-1, keepdims=True)
        i2 = jnp.min(jnp.where(masked == m2, idx8, E), axis=-1, keepdims=True)
        oh1 = (idx8 == i2).astype(F32)

        wsum = m1 + m2
        w_ref[...] = jnp.concatenate([m1 / wsum, m2 / wsum], axis=1)

        ii = lax.broadcasted_iota(jnp.int32, (512, 512), 0)
        jj = lax.broadcasted_iota(jnp.int32, (512, 512), 1)
        tril = (jj < ii).astype(F32)

        def exc_prefix(oh):
            parts = []
            carry = jnp.zeros((1, E), F32)
            for bb in range(4):
                blk = oh[bb * 512:(bb + 1) * 512]
                parts.append(jnp.dot(tril, blk, preferred_element_type=F32,
                                     precision=lax.Precision.HIGHEST) + carry)
                carry = carry + jnp.sum(blk, axis=0, keepdims=True)
            return jnp.concatenate(parts, axis=0), carry

        pre0, c0 = exc_prefix(oh0)
        pre1, c1 = exc_prefix(oh1)
        counts = c0 + c1
        padded = jnp.floor((counts + (BT - 1)) / BT) * BT
        ei = lax.broadcasted_iota(jnp.int32, (E, E), 0)
        ej = lax.broadcasted_iota(jnp.int32, (E, E), 1)
        tril8 = (ei < ej).astype(F32)
        pstart = jnp.dot(padded, tril8, preferred_element_type=F32,
                         precision=lax.Precision.HIGHEST)

        d0 = jnp.sum((pstart + pre0) * oh0, axis=-1, keepdims=True)
        d1 = jnp.sum((pstart + c0 + pre1) * oh1, axis=-1, keepdims=True)
        d0_ref[...] = d0.astype(jnp.int32)
        d1_ref[...] = d1.astype(jnp.int32)

        ts = lax.broadcasted_iota(jnp.int32, (NTILES, E), 0).astype(F32) * BT
        cmp = (ts >= pstart).astype(jnp.int32)
        eot_ref[...] = jnp.sum(cmp, axis=-1, keepdims=True) - 1

    return pl.pallas_call(
        body,
        grid=(1,),
        in_specs=[
            pl.BlockSpec((S, D), lambda i: (0, 0)),
            pl.BlockSpec((D, D), lambda i: (0, 0)),
            pl.BlockSpec((1, D), lambda i: (0, 0)),
            pl.BlockSpec((S, D), lambda i: (0, 0)),
            pl.BlockSpec((1, D), lambda i: (0, 0)),
            pl.BlockSpec((1, D), lambda i: (0, 0)),
            pl.BlockSpec((E, D), lambda i: (0, 0)),
            pl.BlockSpec((1, E), lambda i: (0, 0)),
        ],
        out_specs=[
            pl.BlockSpec((S, D), lambda i: (0, 0)),
            pl.BlockSpec((S, 2), lambda i: (0, 0)),
            pl.BlockSpec((S, 1), lambda i: (0, 0)),
            pl.BlockSpec((S, 1), lambda i: (0, 0)),
            pl.BlockSpec((NTILES, 1), lambda i: (0, 0)),
        ],
        out_shape=[
            jax.ShapeDtypeStruct((S, D), F32),
            jax.ShapeDtypeStruct((S, 2), F32),
            jax.ShapeDtypeStruct((S, 1), jnp.int32),
            jax.ShapeDtypeStruct((S, 1), jnp.int32),
            jax.ShapeDtypeStruct((NTILES, 1), jnp.int32),
        ],
    )(ao, Wo, bo.reshape(1, D), xp, lg.reshape(1, D), lb.reshape(1, D), Wr,
      br.reshape(1, E))


def _gmm(eot, xs, W1l, b1l, W2l, b2l):
    HB = 512

    def body(eot_ref, xs_ref, w1_ref, b1_ref, w2_ref, b2_ref, o_ref):
        j = pl.program_id(1)
        h = jnp.maximum(_dotT(xs_ref[...], w1_ref[0]) + b1_ref[0], 0.0)
        part = _dotT(h, w2_ref[0])

        @pl.when(j == 0)
        def _():
            o_ref[...] = part + b2_ref[0]

        @pl.when(j > 0)
        def _():
            o_ref[...] += part

    grid_spec = pltpu.PrefetchScalarGridSpec(
        num_scalar_prefetch=1,
        grid=(NTILES, HID // HB),
        in_specs=[
            pl.BlockSpec((BT, D), lambda i, j, eot: (i, 0)),
            pl.BlockSpec((1, HB, D), lambda i, j, eot: (eot[i, 0], j, 0)),
            pl.BlockSpec((1, 1, HB), lambda i, j, eot: (eot[i, 0], 0, j)),
            pl.BlockSpec((1, D, HB), lambda i, j, eot: (eot[i, 0], 0, j)),
            pl.BlockSpec((1, 1, D), lambda i, j, eot: (eot[i, 0], 0, 0)),
        ],
        out_specs=pl.BlockSpec((BT, D), lambda i, j, eot: (i, 0)),
    )
    return pl.pallas_call(
        body,
        grid_spec=grid_spec,
        out_shape=jax.ShapeDtypeStruct((NPAD, D), F32),
    )(eot, xs, W1l, b1l.reshape(E, 1, HID), W2l, b2l.reshape(E, 1, D))


def _ln2final(x1, g0, g1, w, lg, lb, Wout, bout):
    RB, CB = 512, 1280

    def body(x_ref, g0_ref, g1_ref, w_ref, lg_ref, lb_ref, wo_ref, b_ref,
             o_ref, x2_scr):
        j = pl.program_id(1)

        @pl.when(j == 0)
        def _():
            ww = w_ref[...]
            moe = ww[:, 0:1] * g0_ref[...] + ww[:, 1:2] * g1_ref[...]
            x2_scr[...] = _layer_norm(x_ref[...] + moe, lg_ref[...],
                                      lb_ref[...])

        o_ref[...] = _dotT(x2_scr[...], wo_ref[...]) + b_ref[...]

    return pl.pallas_call(
        body,
        grid=(S // RB, VOCAB // CB),
        in_specs=[
            pl.BlockSpec((RB, D), lambda i, j: (i, 0)),
            pl.BlockSpec((RB, D), lambda i, j: (i, 0)),
            pl.BlockSpec((RB, D), lambda i, j: (i, 0)),
            pl.BlockSpec((RB, 2), lambda i, j: (i, 0)),
            pl.BlockSpec((1, D), lambda i, j: (0, 0)),
            pl.BlockSpec((1, D), lambda i, j: (0, 0)),
            pl.BlockSpec((CB, D), lambda i, j: (j, 0)),
            pl.BlockSpec((1, CB), lambda i, j: (0, j)),
        ],
        out_specs=pl.BlockSpec((RB, CB), lambda i, j: (i, j)),
        out_shape=jax.ShapeDtypeStruct((S, VOCAB), F32),
        scratch_shapes=[pltpu.VMEM((RB, D), F32)],
    )(x1, g0, g1, w, lg.reshape(1, D), lb.reshape(1, D), Wout,
      bout.reshape(1, VOCAB))


def kernel(src, emb, pos, Wqkv, bqkv, Wo, bo, ln1_g, ln1_b, ln2_g, ln2_b,
           Wr, br, W1, b1, W2, b2, Wout, bout):
    Wqkv = Wqkv.astype(BF16)
    Wo = Wo.astype(BF16)
    W1 = W1.astype(BF16)
    W2 = W2.astype(BF16)
    Wout = Wout.astype(BF16)
    srcf = src.reshape(S)
    gath = _sc_gather(emb, srcf)
    pos2d = pos[0, :S, :]
    moe_state = None
    for l in range(L):
        if l == 0:
            xp, qkv = _qkv0(gath, pos2d, Wqkv[l], bqkv[l])
        else:
            x1p, g0p, g1p, wp = moe_state
            xp, qkv = _ln2qkv(x1p, g0p, g1p, wp, ln2_g[l - 1], ln2_b[l - 1],
                              Wqkv[l], bqkv[l])
        ao = _attn(qkv)
        x1, w, d0, d1, eot = _postattn_route(ao, Wo[l], bo[l], xp,
                                             ln1_g[l], ln1_b[l], Wr[l], br[l])
        d0f = d0.reshape(S)
        d1f = d1.reshape(S)
        xs = _sc_dispatch(x1, d0f, d1f)
        ys = _gmm(eot, xs, W1[l], b1[l], W2[l], b2[l])
        g0, g1 = _sc_combine(ys, d0f, d1f)
        moe_state = (x1, g0, g1, w)
    x1, g0, g1, w = moe_state
    out = _ln2final(x1, g0, g1, w, ln2_g[L - 1], ln2_b[L - 1], Wout, bout)
    return out.reshape(1, S, VOCAB)
